# Initial kernel scaffold; baseline (speedup 1.0000x reference)
import jax
import jax.numpy as jnp
from jax.experimental import pallas as pl


def kernel(pc, flow):
    raise NotImplementedError("write your pallas kernel here")


# speedup vs baseline: 27.5444x; 27.5444x over previous
import functools

import jax
import jax.numpy as jnp
from jax.experimental import pallas as pl
from jax.experimental.pallas import tpu as pltpu

_W_KNN = 3.0
_W_BQ = 1.0
_KNN_K = 16
_KNN_R2 = 0.25 * 0.25
_BQ_K = 64
_BQ_R2 = 0.75 * 0.75
_ROWS = 256
_SEARCH_ITERS = 30


def _tile_kernel(pc_rows_ref, flow_rows_ref, pc_t_ref, flow_t_ref,
                 knn_out_ref, ball_out_ref):
    pc_rows = pc_rows_ref[0]
    flow_rows = flow_rows_ref[0]
    pc_t = pc_t_ref[0]
    flow_t = flow_t_ref[0]

    r_dim, _ = pc_rows.shape
    _, n_dim = pc_t.shape

    inner = jnp.dot(pc_rows, pc_t, preferred_element_type=jnp.float32)
    sq_rows = jnp.sum(pc_rows * pc_rows, axis=1, keepdims=True)
    sq_all = jnp.sum(pc_t * pc_t, axis=0, keepdims=True)
    d2 = jnp.maximum(sq_rows - 2.0 * inner + sq_all, 0.0)

    l1 = (jnp.abs(flow_rows[:, 0:1] - flow_t[0:1, :])
          + jnp.abs(flow_rows[:, 1:2] - flow_t[1:2, :])
          + jnp.abs(flow_rows[:, 2:3] - flow_t[2:3, :]))

    iota = jax.lax.broadcasted_iota(jnp.int32, (r_dim, n_dim), 1)

    w = (d2 <= _BQ_R2).astype(jnp.float32)
    cum = w
    shift = 1
    while shift < n_dim:
        shifted = jnp.concatenate(
            [jnp.zeros((r_dim, shift), jnp.float32), cum[:, :-shift]], axis=1)
        cum = cum + shifted
        shift *= 2
    cum_ex = cum - w
    take = w * (cum_ex < float(_BQ_K)).astype(jnp.float32)
    ball_sum = jnp.sum(take * l1, axis=1, keepdims=True)
    cnt_w = jnp.sum(w, axis=1, keepdims=True)
    deficit_b = jnp.maximum(float(_BQ_K) - cnt_w, 0.0)
    first_idx = jnp.min(jnp.where(w > 0.0, iota, n_dim), axis=1,
                        keepdims=True)
    first_mask = w * (iota == first_idx).astype(jnp.float32)
    e_ball = jnp.sum(first_mask * l1, axis=1, keepdims=True)
    ball_partial = jnp.sum(ball_sum + deficit_b * e_ball)

    def body(_, carry):
        lo, hi = carry
        mid = 0.5 * (lo + hi)
        cnt = jnp.sum((d2 <= mid).astype(jnp.float32), axis=1, keepdims=True)
        pred = cnt >= float(_KNN_K)
        return jnp.where(pred, lo, mid), jnp.where(pred, mid, hi)

    lo0 = jnp.zeros((r_dim, 1), jnp.float32)
    hi0 = jnp.full((r_dim, 1), _KNN_R2, jnp.float32)
    _, hi = jax.lax.fori_loop(0, _SEARCH_ITERS, body, (lo0, hi0))

    sel = (d2 <= hi).astype(jnp.float32)
    knn_sum = jnp.sum(sel * l1, axis=1, keepdims=True)
    c_sel = jnp.sum(sel, axis=1, keepdims=True)
    deficit_k = float(_KNN_K) - c_sel
    rmin = jnp.min(d2, axis=1, keepdims=True)
    fidx = jnp.min(jnp.where(d2 == rmin, iota, n_dim), axis=1,
                   keepdims=True)
    fmask = (iota == fidx).astype(jnp.float32)
    e_knn = jnp.sum(fmask * l1, axis=1, keepdims=True)
    knn_partial = jnp.sum(knn_sum + deficit_k * e_knn)

    knn_out_ref[...] = jnp.full(knn_out_ref.shape, knn_partial, jnp.float32)
    ball_out_ref[...] = jnp.full(ball_out_ref.shape, ball_partial, jnp.float32)


@jax.jit
def kernel(pc, flow):
    b_dim, n_dim, _ = pc.shape
    n_tiles = n_dim // _ROWS
    pc_t = pc.transpose(0, 2, 1)
    flow_t = flow.transpose(0, 2, 1)

    knn_p, ball_p = pl.pallas_call(
        _tile_kernel,
        grid=(b_dim, n_tiles),
        in_specs=[
            pl.BlockSpec((1, _ROWS, 3), lambda b, r: (b, r, 0)),
            pl.BlockSpec((1, _ROWS, 3), lambda b, r: (b, r, 0)),
            pl.BlockSpec((1, 3, n_dim), lambda b, r: (b, 0, 0)),
            pl.BlockSpec((1, 3, n_dim), lambda b, r: (b, 0, 0)),
        ],
        out_specs=[
            pl.BlockSpec((1, 1, 128), lambda b, r: (b * n_tiles + r, 0, 0)),
            pl.BlockSpec((1, 1, 128), lambda b, r: (b * n_tiles + r, 0, 0)),
        ],
        out_shape=[
            jax.ShapeDtypeStruct((b_dim * n_tiles, 1, 128), jnp.float32),
            jax.ShapeDtypeStruct((b_dim * n_tiles, 1, 128), jnp.float32),
        ],
        compiler_params=pltpu.CompilerParams(
            dimension_semantics=("parallel", "parallel")),
    )(pc, flow, pc_t, flow_t)

    knn_total = jnp.sum(knn_p[:, 0, 0])
    ball_total = jnp.sum(ball_p[:, 0, 0])
    denom = float(b_dim * n_dim)
    return (_W_KNN * knn_total / (denom * _KNN_K)
            + _W_BQ * ball_total / (denom * _BQ_K))
